# baseline (device time: 210781 ns/iter reference)
import jax
import jax.numpy as jnp
from jax import lax
from jax.experimental import pallas as pl
from jax.experimental.pallas import tpu as pltpu

B = 4
SQ = 32
SKV = 4096
H = 8
D = 128
HD = H * D
BK = 1024
NKV = SKV // BK
SCALE = D ** -0.5


def _flash_body(q_ref, k_ref, v_ref, o_ref, stats_ref, acc_s, m_s, l_s):
    kv = pl.program_id(1)

    @pl.when(kv == 0)
    def _():
        m_s[...] = jnp.full((SQ, 128), -jnp.inf, jnp.float32)
        l_s[...] = jnp.zeros((SQ, 128), jnp.float32)
        acc_s[...] = jnp.zeros((SQ, HD), jnp.float32)

    for hh in range(H):
        lanes = pl.ds(hh * D, D)
        q = q_ref[0, :, lanes]
        k = k_ref[0, :, lanes]
        v = v_ref[0, :, lanes]

        s = lax.dot_general(
            q, k, (((1,), (1,)), ((), ())), preferred_element_type=jnp.float32
        ) * SCALE

        m_prev = m_s[:, hh:hh + 1]
        m_blk = jnp.max(s, axis=1, keepdims=True)
        m_new = jnp.maximum(m_prev, m_blk)
        alpha = jnp.exp(m_prev - m_new)
        p = jnp.exp(s - m_new)

        l_new = alpha * l_s[:, hh:hh + 1] + jnp.sum(p, axis=1, keepdims=True)
        pv = lax.dot_general(
            p, v, (((1,), (0,)), ((), ())), preferred_element_type=jnp.float32
        )
        acc_s[:, lanes] = alpha * acc_s[:, lanes] + pv
        m_s[:, hh:hh + 1] = m_new
        l_s[:, hh:hh + 1] = l_new

    @pl.when(kv == NKV - 1)
    def _():
        o_ref[0, :, :] = acc_s[...]
        for hh in range(H):
            stats_ref[0, :, 2 * hh:2 * hh + 1] = m_s[:, hh:hh + 1]
            stats_ref[0, :, 2 * hh + 1:2 * hh + 2] = l_s[:, hh:hh + 1]


def _comm_body(o_ref, stats_ref, out_ref, recv_o, recv_stats, send_sems, recv_sems):
    my_x = lax.axis_index("x")
    my_y = lax.axis_index("y")
    nbr = (my_x, 1 - my_y)

    barrier_sem = pltpu.get_barrier_semaphore()
    pl.semaphore_signal(
        barrier_sem, inc=1, device_id=nbr, device_id_type=pl.DeviceIdType.MESH
    )
    pl.semaphore_wait(barrier_sem, 1)

    rdma_o = pltpu.make_async_remote_copy(
        src_ref=o_ref,
        dst_ref=recv_o,
        send_sem=send_sems.at[0],
        recv_sem=recv_sems.at[0],
        device_id=nbr,
        device_id_type=pl.DeviceIdType.MESH,
    )
    rdma_s = pltpu.make_async_remote_copy(
        src_ref=stats_ref,
        dst_ref=recv_stats,
        send_sem=send_sems.at[1],
        recv_sem=recv_sems.at[1],
        device_id=nbr,
        device_id_type=pl.DeviceIdType.MESH,
    )
    rdma_o.start()
    rdma_s.start()
    rdma_o.wait()
    rdma_s.wait()

    for hh in range(H):
        lanes = pl.ds(hh * D, D)
        m_a = stats_ref[:, :, 2 * hh:2 * hh + 1]
        l_a = stats_ref[:, :, 2 * hh + 1:2 * hh + 2]
        m_b = recv_stats[:, :, 2 * hh:2 * hh + 1]
        l_b = recv_stats[:, :, 2 * hh + 1:2 * hh + 2]
        m_g = jnp.maximum(m_a, m_b)
        ea = jnp.exp(m_a - m_g)
        eb = jnp.exp(m_b - m_g)
        denom = ea * l_a + eb * l_b
        out_ref[:, :, lanes] = (
            ea * o_ref[:, :, lanes] + eb * recv_o[:, :, lanes]
        ) / denom


def kernel(Q, K, V):
    Qr = Q.reshape(B, SQ, HD)
    Kr = K.reshape(B, SKV, HD)
    Vr = V.reshape(B, SKV, HD)

    o_un, stats = pl.pallas_call(
        _flash_body,
        grid=(B, NKV),
        in_specs=[
            pl.BlockSpec((1, SQ, HD), lambda b, kv: (b, 0, 0)),
            pl.BlockSpec((1, BK, HD), lambda b, kv: (b, kv, 0)),
            pl.BlockSpec((1, BK, HD), lambda b, kv: (b, kv, 0)),
        ],
        out_specs=[
            pl.BlockSpec((1, SQ, HD), lambda b, kv: (b, 0, 0)),
            pl.BlockSpec((1, SQ, 2 * H), lambda b, kv: (b, 0, 0)),
        ],
        out_shape=[
            jax.ShapeDtypeStruct((B, SQ, HD), jnp.float32),
            jax.ShapeDtypeStruct((B, SQ, 2 * H), jnp.float32),
        ],
        scratch_shapes=[
            pltpu.VMEM((SQ, HD), jnp.float32),
            pltpu.VMEM((SQ, 128), jnp.float32),
            pltpu.VMEM((SQ, 128), jnp.float32),
        ],
    )(Qr, Kr, Vr)

    out = pl.pallas_call(
        _comm_body,
        in_specs=[
            pl.BlockSpec(memory_space=pltpu.VMEM),
            pl.BlockSpec(memory_space=pltpu.VMEM),
        ],
        out_specs=pl.BlockSpec(memory_space=pltpu.VMEM),
        out_shape=jax.ShapeDtypeStruct((B, SQ, HD), jnp.float32),
        scratch_shapes=[
            pltpu.VMEM((B, SQ, HD), jnp.float32),
            pltpu.VMEM((B, SQ, 2 * H), jnp.float32),
            pltpu.SemaphoreType.DMA((2,)),
            pltpu.SemaphoreType.DMA((2,)),
        ],
        compiler_params=pltpu.CompilerParams(collective_id=0),
    )(o_un, stats)
    return out.reshape(B, SQ, H, D)
